# baseline (speedup 1.0000x reference)
import functools

import jax
import jax.numpy as jnp
from jax import lax
from jax.experimental import pallas as pl
from jax.experimental.pallas import tpu as pltpu
from jax.experimental.pallas import tpu_sc as plsc

EMBED_DIM = 64
BATCH = 16384
NUM_CORES = 2
NUM_SUBCORES = 16
NUM_WORKERS = NUM_CORES * NUM_SUBCORES
B_PER_W = BATCH // NUM_WORKERS
N_SEMS = 8


def _gather_body(idx_hbm, table_hbm, out_hbm, idx_v, rows_s, sems, osem):
    sid = lax.axis_index("s")
    wid = sid * NUM_CORES + lax.axis_index("c")
    pltpu.sync_copy(idx_hbm.at[wid], idx_v)
    copies = []
    for g in range(B_PER_W // 16):
        vec = idx_v[pl.ds(g * 16, 16)]
        for k in range(16):
            j = g * 16 + k
            copies.append(
                pltpu.make_async_copy(
                    table_hbm.at[pl.ds(vec[k], 1)],
                    rows_s.at[sid, pl.ds(j, 1)],
                    sems[j % N_SEMS],
                )
            )
    for cp in copies:
        cp.start()
    for cp in copies:
        cp.wait()
    pltpu.make_async_copy(rows_s.at[sid], out_hbm.at[wid], osem).start()
    pltpu.make_async_copy(rows_s.at[sid], out_hbm.at[wid], osem).wait()


@functools.partial(
    pl.kernel,
    out_type=jax.ShapeDtypeStruct(
        (NUM_WORKERS, B_PER_W, EMBED_DIM), jnp.float32
    ),
    mesh=plsc.VectorSubcoreMesh(core_axis_name="c", subcore_axis_name="s"),
    scratch_types=[
        pltpu.VMEM((B_PER_W,), jnp.int32),
        pltpu.VMEM_SHARED((NUM_SUBCORES, B_PER_W, EMBED_DIM), jnp.float32),
        [pltpu.SemaphoreType.DMA] * N_SEMS,
        pltpu.SemaphoreType.DMA,
    ],
)
def _gather_kernel(idx_hbm, table_hbm, out_hbm, idx_v, rows_s, sems, osem):
    _gather_body(idx_hbm, table_hbm, out_hbm, idx_v, rows_s, sems, osem)


def kernel(input_, weight):
    idx = input_.astype(jnp.int32).reshape(NUM_WORKERS, B_PER_W)
    out = _gather_kernel(idx, weight)
    return out.reshape(BATCH, EMBED_DIM)
